# Initial kernel scaffold; baseline (speedup 1.0000x reference)
import jax
import jax.numpy as jnp
from jax.experimental import pallas as pl


def kernel(x, pos, Wq, Wk, Wv, pos_w1, bn1_g, bn1_b, pos_w2, attn_w1, bn2_g, bn2_b, attn_w2, Wf, bf):
    raise NotImplementedError("write your pallas kernel here")


# speedup vs baseline: 1.5006x; 1.5006x over previous
import functools

import jax
import jax.numpy as jnp
import numpy as np
from jax.experimental import pallas as pl
from jax.experimental.pallas import tpu as pltpu

B, N, D, K = 2, 4096, 256, 16
M = B * N
R = M * K
RB = 2048
PB = RB // K
NBLK = R // RB

_HI = jax.lax.Precision.HIGHEST


def _dot(a, b):
    return jax.lax.dot(a, b, precision=_HI, preferred_element_type=jnp.float32)


def _p0_body(x_ref, wq_ref, wk_ref, wv_ref, q_ref, k_ref, v_ref):
    xb = x_ref[...]
    q_ref[...] = _dot(xb, wq_ref[...])
    k_ref[...] = _dot(xb, wk_ref[...])
    v_ref[...] = _dot(xb, wv_ref[...])


def _projections(xf, Wq, Wk, Wv):
    blk = 1024
    grid = (M // blk,)
    spec_x = pl.BlockSpec((blk, D), lambda i: (i, 0))
    spec_w = pl.BlockSpec((D, D), lambda i: (0, 0))
    out = pl.pallas_call(
        _p0_body,
        grid=grid,
        in_specs=[spec_x, spec_w, spec_w, spec_w],
        out_specs=[spec_x, spec_x, spec_x],
        out_shape=[jax.ShapeDtypeStruct((M, D), jnp.float32)] * 3,
        compiler_params=pltpu.CompilerParams(
            dimension_semantics=("parallel",)),
    )(xf, Wq, Wk, Wv)
    return out


def _p1_body(rel_ref, w1_ref, st_ref):
    i = pl.program_id(0)
    pe1 = _dot(rel_ref[...], w1_ref[...])
    s = jnp.sum(pe1, axis=0)
    q2 = jnp.sum(pe1 * pe1, axis=0)
    blk = jnp.concatenate([s[None, :], q2[None, :]], axis=0)

    @pl.when(i == 0)
    def _():
        st_ref[...] = jnp.zeros_like(st_ref)

    st_ref[0:2, :] += blk


def _bn1_stats(relf, pos_w1):
    blk = 8192
    grid = (R // blk,)
    st = pl.pallas_call(
        _p1_body,
        grid=grid,
        in_specs=[pl.BlockSpec((blk, 3), lambda i: (i, 0)),
                  pl.BlockSpec((3, D), lambda i: (0, 0))],
        out_specs=pl.BlockSpec((8, D), lambda i: (0, 0)),
        out_shape=jax.ShapeDtypeStruct((8, D), jnp.float32),
        compiler_params=pltpu.CompilerParams(
            dimension_semantics=("arbitrary",)),
    )(relf, pos_w1)
    return st


def _p2_body(rel_ref, kf_ref, q_ref, w1_ref, w2_ref, wa1_ref, sc1_ref,
             sh1_ref, a1_ref, pe_ref, st_ref):
    i = pl.program_id(0)
    pe1 = _dot(rel_ref[...], w1_ref[...])
    peR = jnp.maximum(pe1 * sc1_ref[...] + sh1_ref[...], 0.0)
    pe = _dot(peR, w2_ref[...])
    qb = q_ref[...]
    qrep = jnp.broadcast_to(qb[:, None, :], (PB, K, D)).reshape(RB, D)
    t = qrep - kf_ref[...] + pe
    a1 = _dot(t, wa1_ref[...])
    a1_ref[...] = a1.astype(jnp.bfloat16)
    pe_ref[...] = pe.astype(jnp.bfloat16)
    s = jnp.sum(a1, axis=0)
    q2 = jnp.sum(a1 * a1, axis=0)
    blk = jnp.concatenate([s[None, :], q2[None, :]], axis=0)

    @pl.when(i == 0)
    def _():
        st_ref[...] = jnp.zeros_like(st_ref)

    st_ref[0:2, :] += blk


def _phase2(relf, kf, q, pos_w1, pos_w2, attn_w1, sc1, sh1):
    grid = (NBLK,)
    a1, pe, st = pl.pallas_call(
        _p2_body,
        grid=grid,
        in_specs=[
            pl.BlockSpec((RB, 3), lambda i: (i, 0)),
            pl.BlockSpec((RB, D), lambda i: (i, 0)),
            pl.BlockSpec((PB, D), lambda i: (i, 0)),
            pl.BlockSpec((3, D), lambda i: (0, 0)),
            pl.BlockSpec((D, D), lambda i: (0, 0)),
            pl.BlockSpec((D, D), lambda i: (0, 0)),
            pl.BlockSpec((1, D), lambda i: (0, 0)),
            pl.BlockSpec((1, D), lambda i: (0, 0)),
        ],
        out_specs=[
            pl.BlockSpec((RB, D), lambda i: (i, 0)),
            pl.BlockSpec((RB, D), lambda i: (i, 0)),
            pl.BlockSpec((8, D), lambda i: (0, 0)),
        ],
        out_shape=[
            jax.ShapeDtypeStruct((R, D), jnp.bfloat16),
            jax.ShapeDtypeStruct((R, D), jnp.bfloat16),
            jax.ShapeDtypeStruct((8, D), jnp.float32),
        ],
        compiler_params=pltpu.CompilerParams(
            dimension_semantics=("arbitrary",)),
    )(relf, kf, q, pos_w1, pos_w2, attn_w1, sc1, sh1)
    return a1, pe, st


def _p3_body(a1_ref, pe_ref, vf_ref, x_ref, sc2_ref, sh2_ref, wa2_ref,
             wf_ref, bf_ref, out_ref):
    a1 = a1_ref[...].astype(jnp.float32)
    h = jnp.maximum(a1 * sc2_ref[...] + sh2_ref[...], 0.0)
    a2 = _dot(h, wa2_ref[...]) * (1.0 / 16.0)
    a3 = a2.reshape(PB, K, D)
    mx = jnp.max(a3, axis=1, keepdims=True)
    e = jnp.exp(a3 - mx)
    w = e / jnp.sum(e, axis=1, keepdims=True)
    val = vf_ref[...] + pe_ref[...].astype(jnp.float32)
    agg = jnp.sum(w * val.reshape(PB, K, D), axis=1)
    out_ref[...] = _dot(agg, wf_ref[...]) + bf_ref[...] + x_ref[...]


def _phase3(a1, pe, vf, xf, sc2, sh2, attn_w2, Wf, bfr):
    grid = (NBLK,)
    out = pl.pallas_call(
        _p3_body,
        grid=grid,
        in_specs=[
            pl.BlockSpec((RB, D), lambda i: (i, 0)),
            pl.BlockSpec((RB, D), lambda i: (i, 0)),
            pl.BlockSpec((RB, D), lambda i: (i, 0)),
            pl.BlockSpec((PB, D), lambda i: (i, 0)),
            pl.BlockSpec((1, D), lambda i: (0, 0)),
            pl.BlockSpec((1, D), lambda i: (0, 0)),
            pl.BlockSpec((D, D), lambda i: (0, 0)),
            pl.BlockSpec((D, D), lambda i: (0, 0)),
            pl.BlockSpec((1, D), lambda i: (0, 0)),
        ],
        out_specs=pl.BlockSpec((PB, D), lambda i: (i, 0)),
        out_shape=jax.ShapeDtypeStruct((M, D), jnp.float32),
        compiler_params=pltpu.CompilerParams(
            dimension_semantics=("parallel",)),
    )(a1, pe, vf, xf, sc2, sh2, attn_w2, Wf, bfr)
    return out


def kernel(x, pos, Wq, Wk, Wv, pos_w1, bn1_g, bn1_b, pos_w2, attn_w1,
           bn2_g, bn2_b, attn_w2, Wf, bf):
    xf = x.reshape(M, D)

    q, kfull, vfull = _projections(xf, Wq, Wk, Wv)

    sq = jnp.sum(pos * pos, axis=-1)
    d2 = sq[:, :, None] + sq[:, None, :] - 2.0 * jnp.einsum(
        'bic,bjc->bij', pos, pos)
    _, knn_idx = jax.lax.top_k(-d2, K)
    gidx = (knn_idx + (jnp.arange(B) * N)[:, None, None]).reshape(R)
    kf = kfull[gidx]
    vf = vfull[gidx]
    posf = pos.reshape(M, 3)
    knn_xyz = posf[gidx]
    relf = jnp.repeat(posf, K, axis=0) - knn_xyz

    st1 = _bn1_stats(relf, pos_w1)
    cnt = jnp.float32(R)
    mean1 = st1[0] / cnt
    var1 = st1[1] / cnt - mean1 * mean1
    sc1 = (bn1_g / jnp.sqrt(var1 + 1e-5))[None, :]
    sh1 = (bn1_b - mean1 * sc1[0])[None, :]

    a1, pe, st2 = _phase2(relf, kf, q, pos_w1, pos_w2, attn_w1, sc1, sh1)
    mean2 = st2[0] / cnt
    var2 = st2[1] / cnt - mean2 * mean2
    sc2 = (bn2_g / jnp.sqrt(var2 + 1e-5))[None, :]
    sh2 = (bn2_b - mean2 * sc2[0])[None, :]

    out = _phase3(a1, pe, vf, xf, sc2, sh2, attn_w2, Wf, bf[None, :])
    return out.reshape(B, N, D)


# speedup vs baseline: 1.8255x; 1.2165x over previous
import functools

import jax
import jax.numpy as jnp
import numpy as np
from jax.experimental import pallas as pl
from jax.experimental.pallas import tpu as pltpu
from jax.experimental.pallas import tpu_sc as plsc

B, N, D, K = 2, 4096, 256, 16
M = B * N
R = M * K
RB = 2048
PB = RB // K
NBLK = R // RB

def _dot(a, b):
    return jax.lax.dot(a, b, preferred_element_type=jnp.float32)


def _p0_body(x_ref, wq_ref, wk_ref, wv_ref, q_ref, k_ref, v_ref):
    xb = x_ref[...]
    q_ref[...] = _dot(xb, wq_ref[...])
    k_ref[...] = _dot(xb, wk_ref[...])
    v_ref[...] = _dot(xb, wv_ref[...])


def _projections(xf, Wq, Wk, Wv):
    blk = 1024
    grid = (M // blk,)
    spec_x = pl.BlockSpec((blk, D), lambda i: (i, 0))
    spec_w = pl.BlockSpec((D, D), lambda i: (0, 0))
    out = pl.pallas_call(
        _p0_body,
        grid=grid,
        in_specs=[spec_x, spec_w, spec_w, spec_w],
        out_specs=[spec_x, spec_x, spec_x],
        out_shape=[jax.ShapeDtypeStruct((M, D), jnp.float32)] * 3,
        compiler_params=pltpu.CompilerParams(
            dimension_semantics=("parallel",)),
    )(xf, Wq, Wk, Wv)
    return out


def _sc_gather(table, idx, Dt, chunk):
    NW = 32
    per_w = R // NW
    nbody = per_w // (2 * chunk)
    mesh = plsc.VectorSubcoreMesh(core_axis_name="c", subcore_axis_name="s")

    @functools.partial(
        pl.kernel, mesh=mesh,
        out_type=jax.ShapeDtypeStruct((R, Dt), jnp.float32),
        scratch_types=[
            pltpu.VMEM((per_w,), jnp.int32),
            pltpu.VMEM((chunk, Dt), jnp.float32),
            pltpu.VMEM((chunk, Dt), jnp.float32),
            pltpu.SemaphoreType.DMA,
            pltpu.SemaphoreType.DMA,
            pltpu.SemaphoreType.DMA,
            pltpu.SemaphoreType.DMA,
        ])
    def gk(table_hbm, idx_hbm, out_hbm, idx_v, buf0, buf1, gs0, gs1, os0, os1):
        wid = jax.lax.axis_index("s") * 2 + jax.lax.axis_index("c")
        base = wid * per_w
        pltpu.sync_copy(idx_hbm.at[pl.ds(base, per_w)], idx_v)
        bufs = (buf0, buf1)
        gss = (gs0, gs1)
        oss = (os0, os1)

        def body(g, carry):
            c0 = g * 2
            for j in range(2):
                @pl.when(g > 0)
                def _(j=j):
                    pltpu.make_async_copy(
                        bufs[j], out_hbm.at[pl.ds(0, chunk)], oss[j]).wait()
                pltpu.async_copy(
                    table_hbm.at[idx_v.at[pl.ds((c0 + j) * chunk, chunk)]],
                    bufs[j], gss[j])
            for j in range(2):
                pltpu.make_async_copy(
                    table_hbm.at[idx_v.at[pl.ds(0, chunk)]], bufs[j],
                    gss[j]).wait()
                pltpu.async_copy(
                    bufs[j], out_hbm.at[pl.ds(base + (c0 + j) * chunk, chunk)],
                    oss[j])
            return carry

        jax.lax.fori_loop(0, nbody, body, 0)
        for j in range(2):
            pltpu.make_async_copy(
                bufs[j], out_hbm.at[pl.ds(0, chunk)], oss[j]).wait()

    return gk(table, idx)


def _d2_body(pos_ref, post_ref, out_ref):
    pr = pos_ref[0]
    pc = post_ref[0]
    xr, yr, zr = pr[:, 0:1], pr[:, 1:2], pr[:, 2:3]
    xc, yc, zc = pc[0:1, :], pc[1:2, :], pc[2:3, :]
    sqr = xr * xr + yr * yr + zr * zr
    sqc = xc * xc + yc * yc + zc * zc
    out_ref[0] = (sqr + sqc) - 2.0 * _dot(pr, pc)


def _dist2(pospl):
    RB2 = 512
    posb = pospl.reshape(B, N, 128)
    post = jnp.swapaxes(posb, 1, 2)
    out = pl.pallas_call(
        _d2_body,
        grid=(B, N // RB2),
        in_specs=[
            pl.BlockSpec((1, RB2, 128), lambda b, i: (b, i, 0)),
            pl.BlockSpec((1, 128, N), lambda b, i: (b, 0, 0)),
        ],
        out_specs=pl.BlockSpec((1, RB2, N), lambda b, i: (b, i, 0)),
        out_shape=jax.ShapeDtypeStruct((B, N, N), jnp.float32),
        compiler_params=pltpu.CompilerParams(
            dimension_semantics=("parallel", "parallel")),
    )(posb, post)
    return out


def _p1_body(xyz_ref, posp_ref, w1_ref, st_ref):
    i = pl.program_id(0)
    npt = posp_ref.shape[0]
    pp = jnp.broadcast_to(posp_ref[...][:, None, :], (npt, K, 128)
                          ).reshape(npt * K, 128)
    rel = pp - xyz_ref[...]
    pe1 = _dot(rel, w1_ref[...])
    s = jnp.sum(pe1, axis=0)
    q2 = jnp.sum(pe1 * pe1, axis=0)
    blk = jnp.concatenate([s[None, :], q2[None, :]], axis=0)

    @pl.when(i == 0)
    def _():
        st_ref[...] = jnp.zeros_like(st_ref)

    st_ref[0:2, :] += blk


def _bn1_stats(kfg, posp, pos_w1):
    blk = 8192
    grid = (R // blk,)
    st = pl.pallas_call(
        _p1_body,
        grid=grid,
        in_specs=[pl.BlockSpec((blk, 128), lambda i: (i, 2)),
                  pl.BlockSpec((blk // K, 128), lambda i: (i, 0)),
                  pl.BlockSpec((128, D), lambda i: (0, 0))],
        out_specs=pl.BlockSpec((8, D), lambda i: (0, 0)),
        out_shape=jax.ShapeDtypeStruct((8, D), jnp.float32),
        compiler_params=pltpu.CompilerParams(
            dimension_semantics=("arbitrary",)),
    )(kfg, posp, pos_w1)
    return st


def _p2_body(kfg_ref, posp_ref, q_ref, w1_ref, w2_ref, wa1_ref,
             sc1_ref, sh1_ref, a1_ref, pe_ref, st_ref):
    i = pl.program_id(0)
    pp = jnp.broadcast_to(posp_ref[...][:, None, :], (PB, K, 128)
                          ).reshape(RB, 128)
    kfg = kfg_ref[...]
    rel = pp - kfg[:, D:]
    pe1 = _dot(rel, w1_ref[...])
    peR = jnp.maximum(pe1 * sc1_ref[...] + sh1_ref[...], 0.0)
    pe = _dot(peR, w2_ref[...])
    qb = q_ref[...]
    qrep = jnp.broadcast_to(qb[:, None, :], (PB, K, D)).reshape(RB, D)
    t = qrep - kfg[:, :D] + pe
    a1 = _dot(t, wa1_ref[...])
    a1_ref[...] = a1.astype(jnp.bfloat16)
    pe_ref[...] = pe.astype(jnp.bfloat16)
    s = jnp.sum(a1, axis=0)
    q2 = jnp.sum(a1 * a1, axis=0)
    blk = jnp.concatenate([s[None, :], q2[None, :]], axis=0)

    @pl.when(i == 0)
    def _():
        st_ref[...] = jnp.zeros_like(st_ref)

    st_ref[0:2, :] += blk


def _phase2(kfg, posp, q, pos_w1, pos_w2, attn_w1, sc1, sh1):
    grid = (NBLK,)
    a1, pe, st = pl.pallas_call(
        _p2_body,
        grid=grid,
        in_specs=[
            pl.BlockSpec((RB, D + 128), lambda i: (i, 0)),
            pl.BlockSpec((PB, 128), lambda i: (i, 0)),
            pl.BlockSpec((PB, D), lambda i: (i, 0)),
            pl.BlockSpec((128, D), lambda i: (0, 0)),
            pl.BlockSpec((D, D), lambda i: (0, 0)),
            pl.BlockSpec((D, D), lambda i: (0, 0)),
            pl.BlockSpec((1, D), lambda i: (0, 0)),
            pl.BlockSpec((1, D), lambda i: (0, 0)),
        ],
        out_specs=[
            pl.BlockSpec((RB, D), lambda i: (i, 0)),
            pl.BlockSpec((RB, D), lambda i: (i, 0)),
            pl.BlockSpec((8, D), lambda i: (0, 0)),
        ],
        out_shape=[
            jax.ShapeDtypeStruct((R, D), jnp.bfloat16),
            jax.ShapeDtypeStruct((R, D), jnp.bfloat16),
            jax.ShapeDtypeStruct((8, D), jnp.float32),
        ],
        compiler_params=pltpu.CompilerParams(
            dimension_semantics=("arbitrary",)),
    )(kfg, posp, q, pos_w1, pos_w2, attn_w1, sc1, sh1)
    return a1, pe, st


def _p3_body(a1_ref, pe_ref, vf_ref, x_ref, sc2_ref, sh2_ref, wa2_ref,
             wf_ref, bf_ref, out_ref):
    a1 = a1_ref[...].astype(jnp.float32)
    h = jnp.maximum(a1 * sc2_ref[...] + sh2_ref[...], 0.0)
    a2 = _dot(h, wa2_ref[...]) * (1.0 / 16.0)
    a3 = a2.reshape(PB, K, D)
    mx = jnp.max(a3, axis=1, keepdims=True)
    e = jnp.exp(a3 - mx)
    w = e / jnp.sum(e, axis=1, keepdims=True)
    val = vf_ref[...] + pe_ref[...].astype(jnp.float32)
    agg = jnp.sum(w * val.reshape(PB, K, D), axis=1)
    out_ref[...] = _dot(agg, wf_ref[...]) + bf_ref[...] + x_ref[...]


def _phase3(a1, pe, vf, xf, sc2, sh2, attn_w2, Wf, bfr):
    grid = (NBLK,)
    out = pl.pallas_call(
        _p3_body,
        grid=grid,
        in_specs=[
            pl.BlockSpec((RB, D), lambda i: (i, 0)),
            pl.BlockSpec((RB, D), lambda i: (i, 0)),
            pl.BlockSpec((RB, D), lambda i: (i, 0)),
            pl.BlockSpec((PB, D), lambda i: (i, 0)),
            pl.BlockSpec((1, D), lambda i: (0, 0)),
            pl.BlockSpec((1, D), lambda i: (0, 0)),
            pl.BlockSpec((D, D), lambda i: (0, 0)),
            pl.BlockSpec((D, D), lambda i: (0, 0)),
            pl.BlockSpec((1, D), lambda i: (0, 0)),
        ],
        out_specs=pl.BlockSpec((PB, D), lambda i: (i, 0)),
        out_shape=jax.ShapeDtypeStruct((M, D), jnp.float32),
        compiler_params=pltpu.CompilerParams(
            dimension_semantics=("parallel",)),
    )(a1, pe, vf, xf, sc2, sh2, attn_w2, Wf, bfr)
    return out


def kernel(x, pos, Wq, Wk, Wv, pos_w1, bn1_g, bn1_b, pos_w2, attn_w1,
           bn2_g, bn2_b, attn_w2, Wf, bf):
    xf = x.reshape(M, D)

    q, kfull, vfull = _projections(xf, Wq, Wk, Wv)

    posf = pos.reshape(M, 3)
    pospl = jnp.zeros((M, 128), jnp.float32).at[:, :3].set(posf)
    d2 = _dist2(pospl)
    _, knn_idx = jax.lax.top_k(-d2, K)
    gidx = (knn_idx + (jnp.arange(B) * N)[:, None, None]).reshape(R)
    kaug = jnp.concatenate([kfull, pospl], axis=1)
    w1p = jnp.zeros((128, D), jnp.float32).at[:3].set(pos_w1)
    kfg = _sc_gather(kaug, gidx, D + 128, 128)
    vf = _sc_gather(vfull, gidx, D, 128)

    st1 = _bn1_stats(kfg, pospl, w1p)
    cnt = jnp.float32(R)
    mean1 = st1[0] / cnt
    var1 = st1[1] / cnt - mean1 * mean1
    sc1 = (bn1_g / jnp.sqrt(var1 + 1e-5))[None, :]
    sh1 = (bn1_b - mean1 * sc1[0])[None, :]

    a1, pe, st2 = _phase2(kfg, pospl, q, w1p, pos_w2, attn_w1, sc1, sh1)
    mean2 = st2[0] / cnt
    var2 = st2[1] / cnt - mean2 * mean2
    sc2 = (bn2_g / jnp.sqrt(var2 + 1e-5))[None, :]
    sh2 = (bn2_b - mean2 * sc2[0])[None, :]

    out = _phase3(a1, pe, vf, xf, sc2, sh2, attn_w2, Wf, bf[None, :])
    return out.reshape(B, N, D)
